# baseline (speedup 1.0000x reference)
import functools

import jax
import jax.numpy as jnp
from jax import lax
from jax.experimental import pallas as pl
from jax.experimental.pallas import tpu as pltpu
from jax.experimental.pallas import tpu_sc as plsc

B = 4
S = 4096
D = 768
N_ROWS = B * S
NC = 2
NS = 16
NW = NC * NS
R_PER_W = N_ROWS // NW
CHUNK = 64
N_CHUNKS = R_PER_W // CHUNK
LANES = 16
D_VECS = D // LANES
SCALE = 27.712812921102035


def _sc_body(idx_hbm, pe_hbm, table_hbm, out_hbm, idx_v, rows_v, pe_v, gsem, psem):
    wid = lax.axis_index("s") * NC + lax.axis_index("c")
    base = wid * R_PER_W
    s0 = lax.rem(base, S)

    pltpu.sync_copy(idx_hbm.at[pl.ds(base, R_PER_W)], idx_v)

    for c in range(N_CHUNKS):
        off = c * CHUNK
        g = pltpu.async_copy(
            table_hbm.at[idx_v.at[pl.ds(off, CHUNK)]], rows_v, gsem)
        p = pltpu.async_copy(
            pe_hbm.at[pl.ds(s0 + off, CHUNK)], pe_v, psem)
        g.wait()
        p.wait()

        def row_body(r, _):
            for d in range(D_VECS):
                sl = pl.ds(d * LANES, LANES)
                rows_v[r, sl] = rows_v[r, sl] * SCALE + pe_v[r, sl]
            return 0

        lax.fori_loop(0, CHUNK, row_body, 0)
        pltpu.sync_copy(rows_v, out_hbm.at[pl.ds(base + off, CHUNK)])


@jax.jit
def _embed(idx, pe, table):
    mesh = plsc.VectorSubcoreMesh(core_axis_name="c", subcore_axis_name="s")
    k = functools.partial(
        pl.kernel,
        mesh=mesh,
        out_type=jax.ShapeDtypeStruct((N_ROWS, D), jnp.float32),
        scratch_types=[
            pltpu.VMEM((R_PER_W,), jnp.int32),
            pltpu.VMEM((CHUNK, D), jnp.float32),
            pltpu.VMEM((CHUNK, D), jnp.float32),
            pltpu.SemaphoreType.DMA,
            pltpu.SemaphoreType.DMA,
        ],
    )(_sc_body)
    return k(idx, pe, table)


def kernel(x, token_table, pe):
    idx = x.reshape(N_ROWS).astype(jnp.int32)
    out = _embed(idx, pe, token_table)
    return out.reshape(B, S, D)


# speedup vs baseline: 1.2735x; 1.2735x over previous
import functools

import jax
import jax.numpy as jnp
from jax import lax
from jax.experimental import pallas as pl
from jax.experimental.pallas import tpu as pltpu
from jax.experimental.pallas import tpu_sc as plsc

B = 4
S = 4096
D = 768
N_ROWS = B * S
NC = 2
NS = 16
NW = NC * NS
R_PER_W = N_ROWS // NW
CHUNK = 32
N_CHUNKS = R_PER_W // CHUNK
LANES = 16
D_VECS = D // LANES
SCALE = 27.712812921102035


def _sc_body(idx_hbm, pe_hbm, table_hbm, out_hbm,
             idx_v, r0, r1, o0, o1, o2,
             g0, g1, p0, p1, p2, s0_, s1_, s2_):
    rows = [r0, r1]
    outs = [o0, o1, o2]
    gsem = [g0, g1]
    psem = [p0, p1, p2]
    ssem = [s0_, s1_, s2_]

    wid = lax.axis_index("s") * NC + lax.axis_index("c")
    base = wid * R_PER_W
    pe0 = lax.rem(base, S)

    pltpu.sync_copy(idx_hbm.at[pl.ds(base, R_PER_W)], idx_v)

    def start_gather(c):
        return pltpu.async_copy(
            table_hbm.at[idx_v.at[pl.ds(c * CHUNK, CHUNK)]],
            rows[c % 2], gsem[c % 2])

    def start_pe(c):
        return pltpu.async_copy(
            pe_hbm.at[pl.ds(pe0 + c * CHUNK, CHUNK)],
            outs[c % 3], psem[c % 3])

    g_h = [None, None]
    p_h = [None, None, None]
    s_h = [None, None, None]
    for c in (0, 1):
        g_h[c] = start_gather(c)
        p_h[c] = start_pe(c)

    for c in range(N_CHUNKS):
        rb = c % 2
        ob = c % 3
        g_h[rb].wait()
        p_h[ob].wait()

        def row_body(r, _, _rb=rb, _ob=ob):
            for d in range(D_VECS):
                sl = pl.ds(d * LANES, LANES)
                plsc.addupdate(outs[_ob].at[r, sl], rows[_rb][r, sl] * SCALE)
            return 0

        lax.fori_loop(0, CHUNK, row_body, 0)

        s_h[ob] = pltpu.async_copy(
            outs[ob], out_hbm.at[pl.ds(base + c * CHUNK, CHUNK)], ssem[ob])

        nxt = c + 2
        if nxt < N_CHUNKS:
            g_h[rb] = start_gather(nxt)
            if c >= 1:
                s_h[nxt % 3].wait()
            p_h[nxt % 3] = start_pe(nxt)

    for c in range(max(N_CHUNKS - 3, 0), N_CHUNKS):
        s_h[c % 3].wait()


@jax.jit
def _embed(idx, pe, table):
    mesh = plsc.VectorSubcoreMesh(core_axis_name="c", subcore_axis_name="s")
    k = functools.partial(
        pl.kernel,
        mesh=mesh,
        out_type=jax.ShapeDtypeStruct((N_ROWS, D), jnp.float32),
        scratch_types=[
            pltpu.VMEM((R_PER_W,), jnp.int32),
            pltpu.VMEM((CHUNK, D), jnp.float32),
            pltpu.VMEM((CHUNK, D), jnp.float32),
            pltpu.VMEM((CHUNK, D), jnp.float32),
            pltpu.VMEM((CHUNK, D), jnp.float32),
            pltpu.VMEM((CHUNK, D), jnp.float32),
            pltpu.SemaphoreType.DMA,
            pltpu.SemaphoreType.DMA,
            pltpu.SemaphoreType.DMA,
            pltpu.SemaphoreType.DMA,
            pltpu.SemaphoreType.DMA,
            pltpu.SemaphoreType.DMA,
            pltpu.SemaphoreType.DMA,
            pltpu.SemaphoreType.DMA,
        ],
    )(_sc_body)
    return k(idx, pe, table)


def kernel(x, token_table, pe):
    idx = x.reshape(N_ROWS).astype(jnp.int32)
    out = _embed(idx, pe, token_table)
    return out.reshape(B, S, D)


# speedup vs baseline: 1.4837x; 1.1651x over previous
import functools

import jax
import jax.numpy as jnp
from jax import lax
from jax.experimental import pallas as pl
from jax.experimental.pallas import tpu as pltpu
from jax.experimental.pallas import tpu_sc as plsc

B = 4
S = 4096
D = 768
N_ROWS = B * S
NC = 2
NS = 16
NW = NC * NS
S_PER_W = S // NW
CHUNK = 32
N_PCH = S_PER_W // CHUNK
N_CHUNKS = N_PCH * B
LANES = 16
D_VECS = D // LANES
SCALE = 27.712812921102035


def _sc_body(idx_hbm, pe_hbm, table_hbm, out_hbm,
             idx_v, r0, r1, r2, pv0, pv1,
             g0, g1, g2, p0, p1, s0_, s1_, s2_):
    rows = [r0, r1, r2]
    pes = [pv0, pv1]
    gsem = [g0, g1, g2]
    psem = [p0, p1]
    ssem = [s0_, s1_, s2_]

    wid = lax.axis_index("s") * NC + lax.axis_index("c")
    w0 = wid * S_PER_W

    for b in range(B):
        pltpu.sync_copy(idx_hbm.at[pl.ds(b * S + w0, S_PER_W)],
                        idx_v.at[pl.ds(b * S_PER_W, S_PER_W)])

    def flat_base(t):
        cc, b = t // B, t % B
        return b * S + w0 + cc * CHUNK

    def start_gather(t):
        cc, b = t // B, t % B
        off = b * S_PER_W + cc * CHUNK
        return pltpu.async_copy(
            table_hbm.at[idx_v.at[pl.ds(off, CHUNK)]],
            rows[t % 3], gsem[t % 3])

    def start_pe(cc):
        return pltpu.async_copy(
            pe_hbm.at[pl.ds(w0 + cc * CHUNK, CHUNK)],
            pes[cc % 2], psem[cc % 2])

    g_h = [None, None, None]
    p_h = [None, None]
    s_h = [None, None, None]
    g_h[0] = start_gather(0)
    g_h[1] = start_gather(1)
    p_h[0] = start_pe(0)

    for t in range(N_CHUNKS):
        cc, b = t // B, t % B
        rb = t % 3
        if b == 0:
            p_h[cc % 2].wait()
        g_h[rb].wait()

        def row_body(r, _, _rb=rb, _pb=cc % 2):
            for d in range(D_VECS):
                sl = pl.ds(d * LANES, LANES)
                rows[_rb][r, sl] = rows[_rb][r, sl] * SCALE + pes[_pb][r, sl]
            return 0

        lax.fori_loop(0, CHUNK, row_body, 0)

        s_h[rb] = pltpu.async_copy(
            rows[rb], out_hbm.at[pl.ds(flat_base(t), CHUNK)], ssem[rb])

        nxt = t + 2
        if nxt < N_CHUNKS:
            if t >= 1:
                s_h[nxt % 3].wait()
            g_h[nxt % 3] = start_gather(nxt)
        if b == 0 and cc + 1 < N_PCH:
            p_h[(cc + 1) % 2] = start_pe(cc + 1)

    for t in range(N_CHUNKS - 3, N_CHUNKS):
        s_h[t % 3].wait()


@jax.jit
def _embed(idx, pe, table):
    mesh = plsc.VectorSubcoreMesh(core_axis_name="c", subcore_axis_name="s")
    k = functools.partial(
        pl.kernel,
        mesh=mesh,
        out_type=jax.ShapeDtypeStruct((N_ROWS, D), jnp.float32),
        scratch_types=[
            pltpu.VMEM((B * S_PER_W,), jnp.int32),
            pltpu.VMEM((CHUNK, D), jnp.float32),
            pltpu.VMEM((CHUNK, D), jnp.float32),
            pltpu.VMEM((CHUNK, D), jnp.float32),
            pltpu.VMEM((CHUNK, D), jnp.float32),
            pltpu.VMEM((CHUNK, D), jnp.float32),
            pltpu.SemaphoreType.DMA,
            pltpu.SemaphoreType.DMA,
            pltpu.SemaphoreType.DMA,
            pltpu.SemaphoreType.DMA,
            pltpu.SemaphoreType.DMA,
            pltpu.SemaphoreType.DMA,
            pltpu.SemaphoreType.DMA,
            pltpu.SemaphoreType.DMA,
        ],
    )(_sc_body)
    return k(idx, pe, table)


def kernel(x, token_table, pe):
    idx = x.reshape(N_ROWS).astype(jnp.int32)
    out = _embed(idx, pe, token_table)
    return out.reshape(B, S, D)
